# baseline (speedup 1.0000x reference)
import jax
import jax.numpy as jnp
import numpy as np
from jax.experimental import pallas as pl
from jax.experimental.pallas import tpu as pltpu
from jax.sharding import Mesh, PartitionSpec as P

_ROWS = 16
_BN = 1024
_INF_BITS = 0x7F800000


def _i32(x):
    return jax.lax.bitcast_convert_type(x, jnp.int32)


def _f32(x):
    return jax.lax.bitcast_convert_type(x, jnp.float32)


def _tree_min_lanes(x):
    w = x.shape[1]
    while w > 128:
        w //= 2
        x = jnp.minimum(x[:, :w], x[:, w : 2 * w])
    return x


def _chamfer_body(thr_ref, p_ref, gb_ref, gsq_ref, fwd_ref, stat_ref,
                  fwd_scr, bwd_scr):
    m = p_ref.shape[0]
    n = gb_ref.shape[2]
    nj = n // _BN
    nr = m // _ROWS

    bwd_scr[:, :] = jnp.full((8, n), _INF_BITS, jnp.int32)

    def rbody(r, carry):
        base = r * _ROWS
        pch = p_ref[pl.ds(base, _ROWS), :]
        pa = pch[0:8]
        pb = pch[8:16]
        fa = jnp.full((8, _BN), _INF_BITS, jnp.int32)
        fb = jnp.full((8, _BN), _INF_BITS, jnp.int32)
        for j in range(nj):
            sl = pl.ds(j * _BN, _BN)
            g0 = gb_ref[0, :, sl]
            g1 = gb_ref[1, :, sl]
            g2 = gb_ref[2, :, sl]
            s = gsq_ref[:, sl]
            d2a = _i32((pa[:, 3:4] + s)
                       + (pa[:, 0:1] * g0 + pa[:, 1:2] * g1 + pa[:, 2:3] * g2))
            d2b = _i32((pb[:, 3:4] + s)
                       + (pb[:, 0:1] * g0 + pb[:, 1:2] * g1 + pb[:, 2:3] * g2))
            fa = jnp.minimum(fa, d2a)
            fb = jnp.minimum(fb, d2b)
            bwd_scr[:, sl] = jnp.minimum(
                jnp.minimum(bwd_scr[:, sl], d2a), d2b)
        fwd_scr[pl.ds(base, 8), :] = _tree_min_lanes(fa)
        fwd_scr[pl.ds(base + 8, 8), :] = _tree_min_lanes(fb)
        return carry

    jax.lax.fori_loop(0, nr, rbody, 0, unroll=False)

    fwd_ref[:, :] = jnp.min(_f32(fwd_scr[:, :]), axis=1, keepdims=True)

    t0 = thr_ref[0]
    t1 = thr_ref[1]
    bmin = jnp.min(_f32(bwd_scr[:, :]), axis=0, keepdims=True)
    bdist = jnp.sqrt(jnp.maximum(bmin, 0.0))
    bsum = jnp.sum(bdist)
    bc0 = jnp.sum((bdist <= t0).astype(jnp.float32))
    bc1 = jnp.sum((bdist <= t1).astype(jnp.float32))
    lane = jax.lax.broadcasted_iota(jnp.int32, (1, 128), 1)
    stat_ref[:, :] = jnp.where(
        lane == 0, bsum, jnp.where(lane == 1, bc0, jnp.where(lane == 2, bc1, 0.0))
    ).astype(jnp.float32)


def _chamfer_block(p4, gb, gsqb, threshes, interpret=False):
    m = p4.shape[0]
    return pl.pallas_call(
        _chamfer_body,
        out_shape=(
            jax.ShapeDtypeStruct((m, 1), jnp.float32),
            jax.ShapeDtypeStruct((1, 128), jnp.float32),
        ),
        in_specs=[
            pl.BlockSpec(memory_space=pltpu.SMEM),
            pl.BlockSpec(memory_space=pltpu.VMEM),
            pl.BlockSpec(memory_space=pltpu.VMEM),
            pl.BlockSpec(memory_space=pltpu.VMEM),
        ],
        out_specs=(
            pl.BlockSpec(memory_space=pltpu.VMEM),
            pl.BlockSpec(memory_space=pltpu.VMEM),
        ),
        scratch_shapes=[
            pltpu.VMEM((m, 128), jnp.int32),
            pltpu.VMEM((8, gb.shape[2]), jnp.int32),
        ],
        interpret=interpret,
    )(threshes, p4, gb, gsqb)


def kernel(predict_pc, gt_pc, threshes):
    p = jnp.transpose(predict_pc[0], (1, 0))
    m = p.shape[0]
    psq = jnp.sum(p * p, axis=1, keepdims=True)
    p4 = jnp.concatenate([-2.0 * p, psq], axis=1)
    g = gt_pc[0]
    n = g.shape[1]
    gb = jnp.broadcast_to(g[:, None, :], (3, 8, n))
    gsq = jnp.sum(g * g, axis=0, keepdims=True)
    gsqb = jnp.broadcast_to(gsq, (8, n))

    devs = jax.devices()
    ndev = len(devs)
    while ndev > 1 and (n % (ndev * _BN) != 0):
        ndev -= 1
    mesh = Mesh(np.array(devs[:ndev]), ("x",))

    def per_device(thr, p4_, gb_, gsqb_):
        fwd, stat = _chamfer_block(p4_, gb_, gsqb_, thr)
        return fwd, stat

    fwd_parts, stat_parts = jax.shard_map(
        per_device,
        mesh=mesh,
        in_specs=(P(), P(), P(None, None, "x"), P(None, "x")),
        out_specs=(P(None, "x"), P("x", None)),
        check_vma=False,
    )(threshes, p4, gb, gsqb)

    t0 = threshes[0]
    t1 = threshes[1]
    fwd_d2 = jnp.min(fwd_parts, axis=1)
    fdist = jnp.sqrt(jnp.maximum(fwd_d2, 0.0))
    fsum = jnp.sum(fdist)
    fc0 = jnp.sum((fdist <= t0).astype(jnp.float32))
    fc1 = jnp.sum((fdist <= t1).astype(jnp.float32))
    bsum = jnp.sum(stat_parts[:, 0])
    bc0 = jnp.sum(stat_parts[:, 1])
    bc1 = jnp.sum(stat_parts[:, 2])

    mf = jnp.float32(m)
    nf = jnp.float32(n)
    cd = fsum / mf * 0.5 + bsum / nf * 0.5

    def fsc(fc, bc):
        prec = 100.0 / mf * fc
        rec = 100.0 / nf * bc
        return 2.0 * prec * rec / (prec + rec + 1e-8)

    return jnp.stack([cd, fsc(fc0, bc0), fsc(fc1, bc1)])


# speedup vs baseline: 3.1831x; 3.1831x over previous
import jax
import jax.numpy as jnp
import numpy as np
from jax.experimental import pallas as pl
from jax.experimental.pallas import tpu as pltpu
from jax.sharding import Mesh, PartitionSpec as P

_ROWS = 16
_BN = 1024
_INF_BITS = 0x7F800000


def _i32(x):
    return jax.lax.bitcast_convert_type(x, jnp.int32)


def _f32(x):
    return jax.lax.bitcast_convert_type(x, jnp.float32)


def _tree_min_lanes(x):
    w = x.shape[1]
    while w > 128:
        w //= 2
        x = jnp.minimum(x[:, :w], x[:, w : 2 * w])
    return x


def _chamfer_body(thr_ref, p_ref, gb_ref, gsq_ref, fwd_ref, stat_ref,
                  fwd_scr, bwd_scr):
    m = p_ref.shape[0]
    n = gb_ref.shape[2]
    nj = n // _BN
    nr = m // _ROWS

    bwd_scr[:, :] = jnp.full((8, n), _INF_BITS, jnp.int32)

    def rbody(r, carry):
        base = r * _ROWS
        pch = p_ref[pl.ds(base, _ROWS), :]
        pa = pch[0:8]
        pb = pch[8:16]
        fa = jnp.full((8, _BN), _INF_BITS, jnp.int32)
        fb = jnp.full((8, _BN), _INF_BITS, jnp.int32)
        for j in range(nj):
            sl = pl.ds(j * _BN, _BN)
            g0 = gb_ref[0, :, sl]
            g1 = gb_ref[1, :, sl]
            g2 = gb_ref[2, :, sl]
            s = gsq_ref[:, sl]
            d2a = _i32((pa[:, 3:4] + s)
                       + (pa[:, 0:1] * g0 + pa[:, 1:2] * g1 + pa[:, 2:3] * g2))
            d2b = _i32((pb[:, 3:4] + s)
                       + (pb[:, 0:1] * g0 + pb[:, 1:2] * g1 + pb[:, 2:3] * g2))
            fa = jnp.minimum(fa, d2a)
            fb = jnp.minimum(fb, d2b)
            bwd_scr[:, sl] = jnp.minimum(
                jnp.minimum(bwd_scr[:, sl], d2a), d2b)
        fwd_scr[pl.ds(base, 8), :] = _tree_min_lanes(fa)
        fwd_scr[pl.ds(base + 8, 8), :] = _tree_min_lanes(fb)
        return carry

    jax.lax.fori_loop(0, nr, rbody, 0, unroll=False)

    fwd_ref[:, :] = jnp.min(_f32(fwd_scr[:, :]), axis=1, keepdims=True)

    t0 = thr_ref[0]
    t1 = thr_ref[1]
    bmin = jnp.min(_f32(bwd_scr[:, :]), axis=0, keepdims=True)
    bdist = jnp.sqrt(jnp.maximum(bmin, 0.0))
    bsum = jnp.sum(bdist)
    bc0 = jnp.sum((bdist <= t0).astype(jnp.float32))
    bc1 = jnp.sum((bdist <= t1).astype(jnp.float32))
    lane = jax.lax.broadcasted_iota(jnp.int32, (1, 128), 1)
    stat_ref[:, :] = jnp.where(
        lane == 0, bsum, jnp.where(lane == 1, bc0, jnp.where(lane == 2, bc1, 0.0))
    ).astype(jnp.float32)


def _chamfer_block(p4, gb, gsqb, threshes, interpret=False):
    m = p4.shape[0]
    return pl.pallas_call(
        _chamfer_body,
        out_shape=(
            jax.ShapeDtypeStruct((m, 1), jnp.float32),
            jax.ShapeDtypeStruct((1, 128), jnp.float32),
        ),
        in_specs=[
            pl.BlockSpec(memory_space=pltpu.SMEM),
            pl.BlockSpec(memory_space=pltpu.VMEM),
            pl.BlockSpec(memory_space=pltpu.VMEM),
            pl.BlockSpec(memory_space=pltpu.VMEM),
        ],
        out_specs=(
            pl.BlockSpec(memory_space=pltpu.VMEM),
            pl.BlockSpec(memory_space=pltpu.VMEM),
        ),
        scratch_shapes=[
            pltpu.VMEM((m, 128), jnp.int32),
            pltpu.VMEM((8, gb.shape[2]), jnp.int32),
        ],
        interpret=interpret,
    )(threshes, p4, gb, gsqb)


def kernel(predict_pc, gt_pc, threshes):
    p = jnp.transpose(predict_pc[0], (1, 0))
    m = p.shape[0]
    psq = jnp.sum(p * p, axis=1, keepdims=True)
    p4 = jnp.concatenate([-2.0 * p, psq], axis=1)
    g = gt_pc[0]
    n = g.shape[1]
    gb = jnp.broadcast_to(g[:, None, :], (3, 8, n))
    gsq = jnp.sum(g * g, axis=0, keepdims=True)
    gsqb = jnp.broadcast_to(gsq, (8, n))

    fwd_parts, stat_parts = _chamfer_block(p4, gb, gsqb, threshes)

    t0 = threshes[0]
    t1 = threshes[1]
    fwd_d2 = jnp.min(fwd_parts, axis=1)
    fdist = jnp.sqrt(jnp.maximum(fwd_d2, 0.0))
    fsum = jnp.sum(fdist)
    fc0 = jnp.sum((fdist <= t0).astype(jnp.float32))
    fc1 = jnp.sum((fdist <= t1).astype(jnp.float32))
    bsum = jnp.sum(stat_parts[:, 0])
    bc0 = jnp.sum(stat_parts[:, 1])
    bc1 = jnp.sum(stat_parts[:, 2])

    mf = jnp.float32(m)
    nf = jnp.float32(n)
    cd = fsum / mf * 0.5 + bsum / nf * 0.5

    def fsc(fc, bc):
        prec = 100.0 / mf * fc
        rec = 100.0 / nf * bc
        return 2.0 * prec * rec / (prec + rec + 1e-8)

    return jnp.stack([cd, fsc(fc0, bc0), fsc(fc1, bc1)])


# speedup vs baseline: 7.9039x; 2.4831x over previous
import jax
import jax.numpy as jnp
from jax.experimental import pallas as pl
from jax.experimental.pallas import tpu as pltpu

_BLK = 256
_INF_BITS = 0x7F800000


def _i32(x):
    return jax.lax.bitcast_convert_type(x, jnp.int32)


def _f32(x):
    return jax.lax.bitcast_convert_type(x, jnp.float32)


def _chamfer_body(thr_ref, a_ref, b_ref, gsq_ref, psq_ref, out_ref,
                  fwd_scr, bwd_scr):
    m = a_ref.shape[0]
    n = b_ref.shape[1]
    nb = m // _BLK

    bwd_scr[:, :] = jnp.full((8, n), _INF_BITS, jnp.int32)

    def rbody(i, carry):
        base = i * _BLK
        a = a_ref[pl.ds(base, _BLK), :]
        d = jax.lax.dot_general(
            a, b_ref[:, :], (((1,), (0,)), ((), ())),
            preferred_element_type=jnp.float32)
        s = psq_ref[pl.ds(base, _BLK), :] + gsq_ref[:, :]
        d2 = s + d
        fwd_scr[pl.ds(base, _BLK), :] = jnp.min(d2, axis=1, keepdims=True)
        bwd_scr[:, :] = jnp.minimum(
            bwd_scr[:, :],
            _i32(jnp.min(d2.reshape(_BLK // 8, 8, n), axis=0)))
        return carry

    jax.lax.fori_loop(0, nb, rbody, 0, unroll=False)

    t0 = thr_ref[0]
    t1 = thr_ref[1]

    fdist = jnp.sqrt(jnp.maximum(fwd_scr[:, :], 0.0))
    fsum = jnp.sum(fdist)
    fc0 = jnp.sum((fdist <= t0).astype(jnp.float32))
    fc1 = jnp.sum((fdist <= t1).astype(jnp.float32))

    bmin = jnp.min(_f32(bwd_scr[:, :]), axis=0, keepdims=True)
    bdist = jnp.sqrt(jnp.maximum(bmin, 0.0))
    bsum = jnp.sum(bdist)
    bc0 = jnp.sum((bdist <= t0).astype(jnp.float32))
    bc1 = jnp.sum((bdist <= t1).astype(jnp.float32))

    mf = jnp.float32(m)
    nf = jnp.float32(n)
    cd = fsum / mf * 0.5 + bsum / nf * 0.5

    def fsc(fc, bc):
        prec = 100.0 / mf * fc
        rec = 100.0 / nf * bc
        return 2.0 * prec * rec / (prec + rec + 1e-8)

    f0 = fsc(fc0, bc0)
    f1 = fsc(fc1, bc1)
    lane = jax.lax.broadcasted_iota(jnp.int32, (1, 128), 1)
    out_ref[:, :] = jnp.where(
        lane == 0, cd, jnp.where(lane == 1, f0, jnp.where(lane == 2, f1, 0.0))
    ).astype(jnp.float32)


def _chamfer(a, b, gsqb, psq, threshes, interpret=False):
    m = a.shape[0]
    n = b.shape[1]
    return pl.pallas_call(
        _chamfer_body,
        out_shape=jax.ShapeDtypeStruct((1, 128), jnp.float32),
        in_specs=[
            pl.BlockSpec(memory_space=pltpu.SMEM),
            pl.BlockSpec(memory_space=pltpu.VMEM),
            pl.BlockSpec(memory_space=pltpu.VMEM),
            pl.BlockSpec(memory_space=pltpu.VMEM),
            pl.BlockSpec(memory_space=pltpu.VMEM),
        ],
        out_specs=pl.BlockSpec(memory_space=pltpu.VMEM),
        scratch_shapes=[
            pltpu.VMEM((m, 1), jnp.float32),
            pltpu.VMEM((8, n), jnp.int32),
        ],
        interpret=interpret,
    )(threshes, a, b, gsqb, psq)


def kernel(predict_pc, gt_pc, threshes):
    p = jnp.transpose(predict_pc[0], (1, 0))
    a = -2.0 * p
    psq = jnp.sum(p * p, axis=1, keepdims=True)
    b = gt_pc[0]
    n = b.shape[1]
    gsq = jnp.sum(b * b, axis=0, keepdims=True)
    gsqb = jnp.broadcast_to(gsq, (_BLK, n))
    out = _chamfer(a, b, gsqb, psq, threshes)
    return out[0, :3]


# speedup vs baseline: 8.4770x; 1.0725x over previous
import jax
import jax.numpy as jnp
from jax.experimental import pallas as pl
from jax.experimental.pallas import tpu as pltpu

_BLK = 256
_INF_BITS = 0x7F800000


def _i32(x):
    return jax.lax.bitcast_convert_type(x, jnp.int32)


def _f32(x):
    return jax.lax.bitcast_convert_type(x, jnp.float32)


def _chamfer_body(thr_ref, a_ref, b_ref, gsq_ref, psq_ref, out_ref,
                  fwd_scr, bwd_scr):
    m = a_ref.shape[0]
    n = b_ref.shape[1]
    nb = m // _BLK

    bwd_scr[:, :] = jnp.full((8, n), _INF_BITS, jnp.int32)

    def rbody(i, carry):
        base = i * _BLK
        a = a_ref[pl.ds(base, _BLK), :]
        d = jax.lax.dot_general(
            a, b_ref[:, :], (((1,), (0,)), ((), ())),
            preferred_element_type=jnp.float32)
        psq3 = psq_ref[pl.ds(base, _BLK), :].reshape(_BLK // 8, 8, 1)
        s = psq3 + gsq_ref[:, :]
        d2 = s + d.reshape(_BLK // 8, 8, n)
        fwd_scr[pl.ds(base, _BLK), :] = jnp.min(
            d2, axis=2, keepdims=False).reshape(_BLK, 1)
        bwd_scr[:, :] = jnp.minimum(
            bwd_scr[:, :], _i32(jnp.min(d2, axis=0)))
        return carry

    jax.lax.fori_loop(0, nb, rbody, 0, unroll=False)

    t0 = thr_ref[0]
    t1 = thr_ref[1]

    fdist = jnp.sqrt(jnp.maximum(fwd_scr[:, :], 0.0))
    fsum = jnp.sum(fdist)
    fc0 = jnp.sum((fdist <= t0).astype(jnp.float32))
    fc1 = jnp.sum((fdist <= t1).astype(jnp.float32))

    bmin = jnp.min(_f32(bwd_scr[:, :]), axis=0, keepdims=True)
    bdist = jnp.sqrt(jnp.maximum(bmin, 0.0))
    bsum = jnp.sum(bdist)
    bc0 = jnp.sum((bdist <= t0).astype(jnp.float32))
    bc1 = jnp.sum((bdist <= t1).astype(jnp.float32))

    mf = jnp.float32(m)
    nf = jnp.float32(n)
    cd = fsum / mf * 0.5 + bsum / nf * 0.5

    def fsc(fc, bc):
        prec = 100.0 / mf * fc
        rec = 100.0 / nf * bc
        return 2.0 * prec * rec / (prec + rec + 1e-8)

    f0 = fsc(fc0, bc0)
    f1 = fsc(fc1, bc1)
    lane = jax.lax.broadcasted_iota(jnp.int32, (1, 128), 1)
    out_ref[:, :] = jnp.where(
        lane == 0, cd, jnp.where(lane == 1, f0, jnp.where(lane == 2, f1, 0.0))
    ).astype(jnp.float32)


def _chamfer(a, b, gsqb, psq, threshes, interpret=False):
    m = a.shape[0]
    n = b.shape[1]
    return pl.pallas_call(
        _chamfer_body,
        out_shape=jax.ShapeDtypeStruct((1, 128), jnp.float32),
        in_specs=[
            pl.BlockSpec(memory_space=pltpu.SMEM),
            pl.BlockSpec(memory_space=pltpu.VMEM),
            pl.BlockSpec(memory_space=pltpu.VMEM),
            pl.BlockSpec(memory_space=pltpu.VMEM),
            pl.BlockSpec(memory_space=pltpu.VMEM),
        ],
        out_specs=pl.BlockSpec(memory_space=pltpu.VMEM),
        scratch_shapes=[
            pltpu.VMEM((m, 1), jnp.float32),
            pltpu.VMEM((8, n), jnp.int32),
        ],
        interpret=interpret,
    )(threshes, a, b, gsqb, psq)


def kernel(predict_pc, gt_pc, threshes):
    p = jnp.transpose(predict_pc[0], (1, 0))
    a = -2.0 * p
    psq = jnp.sum(p * p, axis=1, keepdims=True)
    b = gt_pc[0]
    n = b.shape[1]
    gsq = jnp.sum(b * b, axis=0, keepdims=True)
    gsqb = jnp.broadcast_to(gsq, (8, n))
    out = _chamfer(a, b, gsqb, psq, threshes)
    return out[0, :3]


# speedup vs baseline: 8.6168x; 1.0165x over previous
import jax
import jax.numpy as jnp
from jax.experimental import pallas as pl
from jax.experimental.pallas import tpu as pltpu

_BLK = 256
_BN = 1024
_INF_BITS = 0x7F800000


def _i32(x):
    return jax.lax.bitcast_convert_type(x, jnp.int32)


def _f32(x):
    return jax.lax.bitcast_convert_type(x, jnp.float32)


def _chamfer_body(thr_ref, a_ref, b_ref, gsq_ref, psq_ref, out_ref,
                  fwd_scr, bwd_scr):
    m = a_ref.shape[0]
    n = b_ref.shape[1]
    nb = m // _BLK

    bwd_scr[:, :] = jnp.full((8, n), _INF_BITS, jnp.int32)

    def rbody(i, carry):
        base = i * _BLK
        a = a_ref[pl.ds(base, _BLK), :]
        psq3 = psq_ref[pl.ds(base, _BLK), :].reshape(_BLK // 8, 8, 1)
        for j in range(n // _BN):
            sl = pl.ds(j * _BN, _BN)
            d = jax.lax.dot_general(
                a, b_ref[:, sl], (((1,), (0,)), ((), ())),
                preferred_element_type=jnp.float32)
            d2 = (d.reshape(_BLK // 8, 8, _BN) + gsq_ref[:, sl]) + psq3
            fwd_scr[pl.ds(base, _BLK), j:j + 1] = jnp.min(
                d2, axis=2, keepdims=False).reshape(_BLK, 1)
            bwd_scr[:, sl] = jnp.minimum(
                bwd_scr[:, sl], _i32(jnp.min(d2, axis=0)))
        return carry

    jax.lax.fori_loop(0, nb, rbody, 0, unroll=False)

    t0 = thr_ref[0]
    t1 = thr_ref[1]

    fmin = jnp.min(fwd_scr[:, :], axis=1, keepdims=True)
    fdist = jnp.sqrt(jnp.maximum(fmin, 0.0))
    fsum = jnp.sum(fdist)
    fc0 = jnp.sum((fdist <= t0).astype(jnp.float32))
    fc1 = jnp.sum((fdist <= t1).astype(jnp.float32))

    bmin = jnp.min(_f32(bwd_scr[:, :]), axis=0, keepdims=True)
    bdist = jnp.sqrt(jnp.maximum(bmin, 0.0))
    bsum = jnp.sum(bdist)
    bc0 = jnp.sum((bdist <= t0).astype(jnp.float32))
    bc1 = jnp.sum((bdist <= t1).astype(jnp.float32))

    mf = jnp.float32(m)
    nf = jnp.float32(n)
    cd = fsum / mf * 0.5 + bsum / nf * 0.5

    def fsc(fc, bc):
        prec = 100.0 / mf * fc
        rec = 100.0 / nf * bc
        return 2.0 * prec * rec / (prec + rec + 1e-8)

    f0 = fsc(fc0, bc0)
    f1 = fsc(fc1, bc1)
    lane = jax.lax.broadcasted_iota(jnp.int32, (1, 128), 1)
    out_ref[:, :] = jnp.where(
        lane == 0, cd, jnp.where(lane == 1, f0, jnp.where(lane == 2, f1, 0.0))
    ).astype(jnp.float32)


def _chamfer(a, b, gsqb, psq, threshes, interpret=False):
    m = a.shape[0]
    n = b.shape[1]
    return pl.pallas_call(
        _chamfer_body,
        out_shape=jax.ShapeDtypeStruct((1, 128), jnp.float32),
        in_specs=[
            pl.BlockSpec(memory_space=pltpu.SMEM),
            pl.BlockSpec(memory_space=pltpu.VMEM),
            pl.BlockSpec(memory_space=pltpu.VMEM),
            pl.BlockSpec(memory_space=pltpu.VMEM),
            pl.BlockSpec(memory_space=pltpu.VMEM),
        ],
        out_specs=pl.BlockSpec(memory_space=pltpu.VMEM),
        scratch_shapes=[
            pltpu.VMEM((m, n // _BN), jnp.float32),
            pltpu.VMEM((8, n), jnp.int32),
        ],
        interpret=interpret,
    )(threshes, a, b, gsqb, psq)


def kernel(predict_pc, gt_pc, threshes):
    p = jnp.transpose(predict_pc[0], (1, 0))
    a = -2.0 * p
    psq = jnp.sum(p * p, axis=1, keepdims=True)
    b = gt_pc[0]
    n = b.shape[1]
    gsq = jnp.sum(b * b, axis=0, keepdims=True)
    gsqb = jnp.broadcast_to(gsq, (8, n))
    out = _chamfer(a, b, gsqb, psq, threshes)
    return out[0, :3]


# speedup vs baseline: 9.4806x; 1.1002x over previous
import jax
import jax.numpy as jnp
from jax.experimental import pallas as pl
from jax.experimental.pallas import tpu as pltpu

_BLK = 256
_BN = 1024
_INF_BITS = 0x7F800000


def _i32(x):
    return jax.lax.bitcast_convert_type(x, jnp.int32)


def _f32(x):
    return jax.lax.bitcast_convert_type(x, jnp.float32)


def _chamfer_body(thr_ref, p_ref, g_ref, out_ref,
                  fwd_scr, bwd_scr, b2_scr, gsq_scr, psq_scr):
    m = p_ref.shape[0]
    n = g_ref.shape[1]
    nb = m // _BLK

    g = g_ref[:, :]
    b2_scr[0:3, :] = -2.0 * g
    gsq = (g[0:1, :] * g[0:1, :] + g[1:2, :] * g[1:2, :]
           + g[2:3, :] * g[2:3, :])
    gsq_scr[:, :] = jnp.broadcast_to(gsq, (8, n))
    pch = p_ref[:, :]
    psq_scr[:, :] = jnp.sum(pch * pch, axis=1, keepdims=True)
    bwd_scr[:, :] = jnp.full((8, n), _INF_BITS, jnp.int32)

    def rbody(i, carry):
        base = i * _BLK
        a = p_ref[pl.ds(base, _BLK), :]
        psq3 = psq_scr[pl.ds(base, _BLK), :].reshape(_BLK // 8, 8, 1)
        for j in range(n // _BN):
            sl = pl.ds(j * _BN, _BN)
            d = jax.lax.dot_general(
                a, b2_scr[0:3, sl], (((1,), (0,)), ((), ())),
                preferred_element_type=jnp.float32)
            d2 = (d.reshape(_BLK // 8, 8, _BN) + gsq_scr[:, sl]) + psq3
            fwd_scr[pl.ds(base, _BLK), j:j + 1] = jnp.min(
                d2, axis=2, keepdims=False).reshape(_BLK, 1)
            bwd_scr[:, sl] = jnp.minimum(
                bwd_scr[:, sl], _i32(jnp.min(d2, axis=0)))
        return carry

    jax.lax.fori_loop(0, nb, rbody, 0, unroll=False)

    t0 = thr_ref[0]
    t1 = thr_ref[1]

    fmin = jnp.min(fwd_scr[:, :], axis=1, keepdims=True)
    fdist = jnp.sqrt(jnp.maximum(fmin, 0.0))
    fsum = jnp.sum(fdist)
    fc0 = jnp.sum((fdist <= t0).astype(jnp.float32))
    fc1 = jnp.sum((fdist <= t1).astype(jnp.float32))

    bmin = jnp.min(_f32(bwd_scr[:, :]), axis=0, keepdims=True)
    bdist = jnp.sqrt(jnp.maximum(bmin, 0.0))
    bsum = jnp.sum(bdist)
    bc0 = jnp.sum((bdist <= t0).astype(jnp.float32))
    bc1 = jnp.sum((bdist <= t1).astype(jnp.float32))

    mf = jnp.float32(m)
    nf = jnp.float32(n)
    cd = fsum / mf * 0.5 + bsum / nf * 0.5

    def fsc(fc, bc):
        prec = 100.0 / mf * fc
        rec = 100.0 / nf * bc
        return 2.0 * prec * rec / (prec + rec + 1e-8)

    f0 = fsc(fc0, bc0)
    f1 = fsc(fc1, bc1)
    lane = jax.lax.broadcasted_iota(jnp.int32, (1, 128), 1)
    out_ref[:, :] = jnp.where(
        lane == 0, cd, jnp.where(lane == 1, f0, jnp.where(lane == 2, f1, 0.0))
    ).astype(jnp.float32)


def _chamfer(pt, g, threshes, interpret=False):
    m = pt.shape[0]
    n = g.shape[1]
    return pl.pallas_call(
        _chamfer_body,
        out_shape=jax.ShapeDtypeStruct((1, 128), jnp.float32),
        in_specs=[
            pl.BlockSpec(memory_space=pltpu.SMEM),
            pl.BlockSpec(memory_space=pltpu.VMEM),
            pl.BlockSpec(memory_space=pltpu.VMEM),
        ],
        out_specs=pl.BlockSpec(memory_space=pltpu.VMEM),
        scratch_shapes=[
            pltpu.VMEM((m, n // _BN), jnp.float32),
            pltpu.VMEM((8, n), jnp.int32),
            pltpu.VMEM((8, n), jnp.float32),
            pltpu.VMEM((8, n), jnp.float32),
            pltpu.VMEM((m, 1), jnp.float32),
        ],
        interpret=interpret,
    )(threshes, pt, g)


def kernel(predict_pc, gt_pc, threshes):
    pt = jnp.transpose(predict_pc[0], (1, 0))
    out = _chamfer(pt, gt_pc[0], threshes)
    return out[0, :3]


# speedup vs baseline: 10.0273x; 1.0577x over previous
import jax
import jax.numpy as jnp
from jax.experimental import pallas as pl
from jax.experimental.pallas import tpu as pltpu

_BLK = 256
_BN = 1024
_INF_BITS = 0x7F800000


def _i32(x):
    return jax.lax.bitcast_convert_type(x, jnp.int32)


def _f32(x):
    return jax.lax.bitcast_convert_type(x, jnp.float32)


def _chamfer_body(thr_ref, p_ref, g_ref, out_ref,
                  fwd_scr, bwd_scr, b2_scr, gsq_scr, psq_scr, pt_scr):
    m = p_ref.shape[1]
    n = g_ref.shape[1]
    nb = m // _BLK

    g = g_ref[:, :]
    b2_scr[0:3, :] = -2.0 * g
    gsq = (g[0:1, :] * g[0:1, :] + g[1:2, :] * g[1:2, :]
           + g[2:3, :] * g[2:3, :])
    gsq_scr[:, :] = jnp.broadcast_to(gsq, (8, n))
    pt_scr[:, :] = jnp.transpose(p_ref[:, :], (1, 0))
    pch = pt_scr[:, :]
    psq_scr[:, :] = jnp.sum(pch * pch, axis=1, keepdims=True)
    bwd_scr[:, :] = jnp.full((8, n), _INF_BITS, jnp.int32)

    def rbody(i, carry):
        base = i * _BLK
        a = pt_scr[pl.ds(base, _BLK), :]
        psq3 = psq_scr[pl.ds(base, _BLK), :].reshape(_BLK // 8, 8, 1)
        for j in range(n // _BN):
            sl = pl.ds(j * _BN, _BN)
            d = jax.lax.dot_general(
                a, b2_scr[0:3, sl], (((1,), (0,)), ((), ())),
                preferred_element_type=jnp.float32)
            d2 = (d.reshape(_BLK // 8, 8, _BN) + gsq_scr[:, sl]) + psq3
            fwd_scr[pl.ds(base, _BLK), j:j + 1] = jnp.min(
                d2, axis=2, keepdims=False).reshape(_BLK, 1)
            bwd_scr[:, sl] = jnp.minimum(
                bwd_scr[:, sl], _i32(jnp.min(d2, axis=0)))
        return carry

    jax.lax.fori_loop(0, nb, rbody, 0, unroll=False)

    t0 = thr_ref[0]
    t1 = thr_ref[1]

    fmin = jnp.min(fwd_scr[:, :], axis=1, keepdims=True)
    fdist = jnp.sqrt(jnp.maximum(fmin, 0.0))
    fsum = jnp.sum(fdist)
    fc0 = jnp.sum((fdist <= t0).astype(jnp.float32))
    fc1 = jnp.sum((fdist <= t1).astype(jnp.float32))

    bmin = jnp.min(_f32(bwd_scr[:, :]), axis=0, keepdims=True)
    bdist = jnp.sqrt(jnp.maximum(bmin, 0.0))
    bsum = jnp.sum(bdist)
    bc0 = jnp.sum((bdist <= t0).astype(jnp.float32))
    bc1 = jnp.sum((bdist <= t1).astype(jnp.float32))

    mf = jnp.float32(m)
    nf = jnp.float32(n)
    cd = fsum / mf * 0.5 + bsum / nf * 0.5

    def fsc(fc, bc):
        prec = 100.0 / mf * fc
        rec = 100.0 / nf * bc
        return 2.0 * prec * rec / (prec + rec + 1e-8)

    f0 = fsc(fc0, bc0)
    f1 = fsc(fc1, bc1)
    lane = jax.lax.broadcasted_iota(jnp.int32, (1, 128), 1)
    out_ref[:, :] = jnp.where(
        lane == 0, cd, jnp.where(lane == 1, f0, jnp.where(lane == 2, f1, 0.0))
    ).astype(jnp.float32)


def _chamfer(pt, g, threshes, interpret=False):
    m = pt.shape[1]
    n = g.shape[1]
    return pl.pallas_call(
        _chamfer_body,
        out_shape=jax.ShapeDtypeStruct((1, 128), jnp.float32),
        in_specs=[
            pl.BlockSpec(memory_space=pltpu.SMEM),
            pl.BlockSpec(memory_space=pltpu.VMEM),
            pl.BlockSpec(memory_space=pltpu.VMEM),
        ],
        out_specs=pl.BlockSpec(memory_space=pltpu.VMEM),
        scratch_shapes=[
            pltpu.VMEM((m, n // _BN), jnp.float32),
            pltpu.VMEM((8, n), jnp.int32),
            pltpu.VMEM((8, n), jnp.float32),
            pltpu.VMEM((8, n), jnp.float32),
            pltpu.VMEM((m, 1), jnp.float32),
            pltpu.VMEM((m, 3), jnp.float32),
        ],
        interpret=interpret,
    )(threshes, pt, g)


def kernel(predict_pc, gt_pc, threshes):
    out = _chamfer(predict_pc[0], gt_pc[0], threshes)
    return out[0, :3]
